# Initial kernel scaffold; baseline (speedup 1.0000x reference)
import jax
import jax.numpy as jnp
from jax.experimental import pallas as pl


def kernel(x, edge_index, fa_w1, fa_b1, fa_w2, fa_b2, mp_w, mp_b, g1_w, g1_al, g1_ar, g1_b, g2_w, g2_al, g2_ar, g2_b, agg_w, agg_b, c_w1, c_b1, c_w2, c_b2):
    raise NotImplementedError("write your pallas kernel here")


# speedup vs baseline: 13.0833x; 13.0833x over previous
import functools

import jax
import jax.numpy as jnp
from jax import lax
from jax.experimental import pallas as pl
from jax.experimental.pallas import tpu as pltpu
from jax.experimental.pallas import tpu_sc as plsc

N = 10000
NP = 10240
AR = 10016
E = 320000
EP = 323584
CH = 128
D = 128
HID = 128
OUT = 64
GW = 200
CHM = 16
TW = 144
NC, NS = 2, 16

_BLK = 1024
_GRID = NP // _BLK

F32 = jnp.float32


def _dot(a, b):
    return jax.lax.dot_general(a, b, (((1,), (0,)), ((), ())),
                               precision=jax.lax.Precision.HIGHEST,
                               preferred_element_type=F32)


def _tc1_body(x_ref, w1_ref, b1_ref, w2_ref, b2_ref, f_ref, g0_ref, g1_ref):
    x = x_ref[...]
    hgate = jnp.maximum(_dot(x, w1_ref[...]) + b1_ref[...], 0.0)
    gates = jax.nn.sigmoid(_dot(hgate, w2_ref[...]) + b2_ref[...])
    f = x * gates
    f2 = f * f
    f3 = f2 * f
    z16 = jnp.zeros((x.shape[0], 16), F32)
    ones1 = jnp.ones((x.shape[0], 1), F32)
    f_ref[...] = f
    g0_ref[...] = jnp.concatenate([f, f2[:, :72]], axis=1)
    g1_ref[...] = jnp.concatenate([f2[:, 72:], f3, ones1, z16[:, :15]], axis=1)


def _tc1(x, fa_w1, fa_b1, fa_w2, fa_b2):
    row = lambda i: (i, 0)
    full = lambda i: (0, 0)
    return pl.pallas_call(
        _tc1_body,
        grid=(_GRID,),
        in_specs=[
            pl.BlockSpec((_BLK, D), row),
            pl.BlockSpec((D, 32), full),
            pl.BlockSpec((1, 32), full),
            pl.BlockSpec((32, D), full),
            pl.BlockSpec((1, D), full),
        ],
        out_specs=[
            pl.BlockSpec((_BLK, D), row),
            pl.BlockSpec((_BLK, GW), row),
            pl.BlockSpec((_BLK, GW), row),
        ],
        out_shape=[
            jax.ShapeDtypeStruct((NP, D), F32),
            jax.ShapeDtypeStruct((NP, GW), F32),
            jax.ShapeDtypeStruct((NP, GW), F32),
        ],
    )(x, fa_w1, fa_b1, fa_w2, fa_b2)


def _tc2_body(s0_ref, s1_ref, f_ref, mpw_ref, mpb_ref, gw_ref, wl_ref, wr_ref,
              t_ref, r_ref):
    s0 = s0_ref[...]
    s1 = s1_ref[...]
    f = f_ref[...]
    cnt = s1[:, 184:185]
    d = jnp.maximum(cnt, 1.0)
    m1 = s0[:, :128] / d
    m2 = jnp.concatenate([s0[:, 128:200], s1[:, :56]], axis=1) / d
    m3 = s1[:, 56:184] / d
    var = jnp.maximum(m2 - m1 * m1, 0.0)
    t = var + 1e-6
    skew = (m3 - 3.0 * m1 * m2 + 2.0 * m1 * m1 * m1) / (t * jnp.sqrt(t) + 1e-6)
    mixed = jnp.concatenate([f, m1, var, skew], axis=1)
    h = jnp.maximum(_dot(mixed, mpw_ref[...]) + mpb_ref[...], 0.0)
    hp = _dot(h, gw_ref[...])
    el16 = _dot(hp, wl_ref[...])
    er16 = _dot(hp, wr_ref[...])
    t_ref[...] = jnp.concatenate([hp, el16], axis=1)
    r_ref[...] = er16


def _tc2(s0, s1, f, mp_w, mp_b, g1_w, wl1, wr1):
    row = lambda i: (i, 0)
    full = lambda i: (0, 0)
    return pl.pallas_call(
        _tc2_body,
        grid=(_GRID,),
        in_specs=[
            pl.BlockSpec((_BLK, GW), row),
            pl.BlockSpec((_BLK, GW), row),
            pl.BlockSpec((_BLK, D), row),
            pl.BlockSpec((4 * D, HID), full),
            pl.BlockSpec((1, HID), full),
            pl.BlockSpec((HID, HID), full),
            pl.BlockSpec((HID, 16), full),
            pl.BlockSpec((HID, 16), full),
        ],
        out_specs=[
            pl.BlockSpec((_BLK, TW), row),
            pl.BlockSpec((_BLK, 16), row),
        ],
        out_shape=[
            jax.ShapeDtypeStruct((NP, TW), F32),
            jax.ShapeDtypeStruct((NP, 16), F32),
        ],
    )(s0, s1, f, mp_w, mp_b, g1_w, wl1, wr1)


def _tc3_body(p0_ref, p1_ref, bh_ref, g1b_ref, gw_ref, wl_ref, wr_ref,
              x1_ref, t_ref, r_ref):
    p = p0_ref[...] + p1_ref[...]
    den = _dot(p[:, :16], bh_ref[...])
    x1 = p[:, 16:] / (den + 1e-9) + g1b_ref[...]
    x1 = jnp.where(x1 > 0, x1, jnp.exp(x1) - 1.0)
    hp = _dot(x1, gw_ref[...])
    el16 = _dot(hp, wl_ref[...])
    er16 = _dot(hp, wr_ref[...])
    x1_ref[...] = x1
    t_ref[...] = jnp.concatenate([hp, el16], axis=1)
    r_ref[...] = er16


def _tc3(p0, p1, bh1, g1_b, g2_w, wl2, wr2):
    row = lambda i: (i, 0)
    full = lambda i: (0, 0)
    return pl.pallas_call(
        _tc3_body,
        grid=(_GRID,),
        in_specs=[
            pl.BlockSpec((_BLK, TW), row),
            pl.BlockSpec((_BLK, TW), row),
            pl.BlockSpec((16, HID), full),
            pl.BlockSpec((1, HID), full),
            pl.BlockSpec((HID, HID), full),
            pl.BlockSpec((HID, 16), full),
            pl.BlockSpec((HID, 16), full),
        ],
        out_specs=[
            pl.BlockSpec((_BLK, HID), row),
            pl.BlockSpec((_BLK, TW), row),
            pl.BlockSpec((_BLK, 16), row),
        ],
        out_shape=[
            jax.ShapeDtypeStruct((NP, HID), F32),
            jax.ShapeDtypeStruct((NP, TW), F32),
            jax.ShapeDtypeStruct((NP, 16), F32),
        ],
    )(p0, p1, bh1, g1_b, g2_w, wl2, wr2)


def _tc4_body(q0_ref, q1_ref, x1_ref, bh_ref, g2b_ref, aw1_ref, aw2_ref,
              ab_ref, cw1_ref, cb1_ref, cw2_ref, cb2_ref, out_ref):
    q = q0_ref[...] + q1_ref[...]
    den = _dot(q[:, :16], bh_ref[...])
    x2 = q[:, 16:] / (den + 1e-9) + g2b_ref[...]
    x2 = jnp.where(x2 > 0, x2, jnp.exp(x2) - 1.0)
    x1 = x1_ref[...]
    agg = jnp.maximum(_dot(x1, aw1_ref[...]) + _dot(x2, aw2_ref[...])
                      + ab_ref[...], 0.0)
    hc = jnp.maximum(_dot(agg, cw1_ref[...]) + cb1_ref[...], 0.0)
    out_ref[...] = _dot(hc, cw2_ref[...]) + cb2_ref[...]


def _tc4(q0, q1, x1, bh2, g2_b, aw1, aw2, agg_b, c_w1, c_b1, c_w2, c_b2):
    row = lambda i: (i, 0)
    full = lambda i: (0, 0)
    return pl.pallas_call(
        _tc4_body,
        grid=(_GRID,),
        in_specs=[
            pl.BlockSpec((_BLK, TW), row),
            pl.BlockSpec((_BLK, TW), row),
            pl.BlockSpec((_BLK, HID), row),
            pl.BlockSpec((16, HID), full),
            pl.BlockSpec((1, HID), full),
            pl.BlockSpec((HID, HID), full),
            pl.BlockSpec((HID, HID), full),
            pl.BlockSpec((1, HID), full),
            pl.BlockSpec((HID, HID), full),
            pl.BlockSpec((1, HID), full),
            pl.BlockSpec((HID, OUT), full),
            pl.BlockSpec((1, OUT), full),
        ],
        out_specs=[pl.BlockSpec((_BLK, OUT), row)],
        out_shape=[jax.ShapeDtypeStruct((NP, OUT), F32)],
    )(q0, q1, x1, bh2, g2_b, aw1, aw2, agg_b, c_w1, c_b1, c_w2, c_b2)[0]


def _zero_vmem(buf, rows, cols):
    z = jnp.zeros((16,), F32)

    def row_body(r, _):
        def col_body(j, _):
            buf[r, pl.ds(j * 16, 16)] = z
            return 0
        return lax.fori_loop(0, cols // 16, col_body, 0)

    lax.fori_loop(0, rows, row_body, 0)


def _zero_acc(acc, zbuf, wid, zr):
    nchk = AR // zr
    rem = AR - nchk * zr
    for i in range(-(-nchk // NS)):
        cidx = wid + i * NS

        @pl.when(cidx < nchk)
        def _():
            pltpu.sync_copy(zbuf, acc.at[pl.ds(cidx * zr, zr), :])

    if rem:
        @pl.when(wid == 0)
        def _():
            pltpu.sync_copy(zbuf.at[pl.ds(0, rem), :],
                            acc.at[pl.ds(nchk * zr, rem), :])


def _copy_out(acc, out_ref, wid, zr):
    nchk = AR // zr
    rem = AR - nchk * zr
    for i in range(-(-nchk // NS)):
        cidx = wid + i * NS

        @pl.when(cidx < nchk)
        def _():
            pltpu.sync_copy(acc.at[pl.ds(cidx * zr, zr), :],
                            out_ref.at[pl.ds(cidx * zr, zr), :])

    if rem:
        @pl.when(wid == 0)
        def _():
            pltpu.sync_copy(acc.at[pl.ds(nchk * zr, rem), :],
                            out_ref.at[pl.ds(nchk * zr, rem), :])


def _moment_kernel(srcp, dstp, g0, g1, s0_out, s1_out,
                   idx_s, idx_d, rows, acc):
    cid = lax.axis_index("c")
    wid = lax.axis_index("s")

    _zero_vmem(rows, CHM, GW)
    _zero_acc(acc, rows, wid, CHM)
    plsc.subcore_barrier()

    per_sub = EP // NS
    base = wid * per_sub

    def chunk(k, _):
        off = base + k * CHM
        pltpu.sync_copy(srcp.at[pl.ds(off, CHM)], idx_s)
        pltpu.sync_copy(dstp.at[pl.ds(off, CHM)], idx_d)

        @pl.when(cid == 0)
        def _():
            pltpu.sync_copy(g0.at[idx_s], rows)

        @pl.when(cid == 1)
        def _():
            pltpu.sync_copy(g1.at[idx_s], rows)

        pltpu.sync_copy(rows, acc.at[idx_d], add=True)
        return 0

    lax.fori_loop(0, per_sub // CHM, chunk, 0)
    plsc.subcore_barrier()

    @pl.when(cid == 0)
    def _():
        _copy_out(acc, s0_out, wid, CHM)

    @pl.when(cid == 1)
    def _():
        _copy_out(acc, s1_out, wid, CHM)


def _sc_moments(srcp, dstp, g0, g1):
    mesh = plsc.VectorSubcoreMesh(core_axis_name="c", subcore_axis_name="s")
    kfn = pl.kernel(
        _moment_kernel,
        out_type=[
            jax.ShapeDtypeStruct((NP, GW), F32),
            jax.ShapeDtypeStruct((NP, GW), F32),
        ],
        mesh=mesh,
        compiler_params=pltpu.CompilerParams(use_tc_tiling_on_sc=False),
        scratch_types=[
            pltpu.VMEM((CHM,), jnp.int32),
            pltpu.VMEM((CHM,), jnp.int32),
            pltpu.VMEM((CHM, GW), F32),
            pltpu.VMEM_SHARED((AR, GW), F32),
        ],
    )
    return kfn(srcp, dstp, g0, g1)


def _gat_kernel(heads, srcp, dstp, tbl, rtbl, p0_out, p1_out,
                idx_s, idx_d, rbuf, ebuf, obuf, acc):
    cid = lax.axis_index("c")
    wid = lax.axis_index("s")

    _zero_vmem(obuf, CH, TW)
    _zero_acc(acc, obuf, wid, CH)
    plsc.subcore_barrier()

    w32 = wid * NC + cid
    per_w = EP // (NC * NS)
    base = w32 * per_w
    bph = (128 // heads) // 16

    def chunk(k, _):
        off = base + k * CH
        pltpu.sync_copy(srcp.at[pl.ds(off, CH)], idx_s)
        pltpu.sync_copy(dstp.at[pl.ds(off, CH)], idx_d)
        pltpu.sync_copy(tbl.at[idx_s], rbuf)
        pltpu.sync_copy(rtbl.at[idx_d], ebuf)

        def edge(e, _):
            vel = rbuf[e, pl.ds(128, 16)]
            ver = ebuf[e, pl.ds(0, 16)]
            s = vel + ver
            w = jnp.exp(jnp.where(s > 0, s, 0.2 * s))
            obuf[e, pl.ds(0, 16)] = w
            for h in range(heads):
                wh = w[h]
                for b in range(bph):
                    c0 = h * (128 // heads) + b * 16
                    obuf[e, pl.ds(16 + c0, 16)] = rbuf[e, pl.ds(c0, 16)] * wh
            return 0

        lax.fori_loop(0, CH, edge, 0)
        pltpu.sync_copy(obuf, acc.at[idx_d], add=True)
        return 0

    lax.fori_loop(0, per_w // CH, chunk, 0)
    plsc.subcore_barrier()

    @pl.when(cid == 0)
    def _():
        _copy_out(acc, p0_out, wid, CH)

    @pl.when(cid == 1)
    def _():
        _copy_out(acc, p1_out, wid, CH)


def _sc_gat(heads, srcp, dstp, tbl, rtbl):
    mesh = plsc.VectorSubcoreMesh(core_axis_name="c", subcore_axis_name="s")
    kfn = pl.kernel(
        functools.partial(_gat_kernel, heads),
        out_type=[
            jax.ShapeDtypeStruct((NP, TW), F32),
            jax.ShapeDtypeStruct((NP, TW), F32),
        ],
        mesh=mesh,
        compiler_params=pltpu.CompilerParams(use_tc_tiling_on_sc=False),
        scratch_types=[
            pltpu.VMEM((CH,), jnp.int32),
            pltpu.VMEM((CH,), jnp.int32),
            pltpu.VMEM((CH, TW), F32),
            pltpu.VMEM((CH, 16), F32),
            pltpu.VMEM((CH, TW), F32),
            pltpu.VMEM_SHARED((AR, TW), F32),
        ],
    )
    return kfn(srcp, dstp, tbl, rtbl)


def kernel(x, edge_index, fa_w1, fa_b1, fa_w2, fa_b2, mp_w, mp_b,
           g1_w, g1_al, g1_ar, g1_b, g2_w, g2_al, g2_ar, g2_b,
           agg_w, agg_b, c_w1, c_b1, c_w2, c_b2):
    xp = jnp.pad(x, ((0, NP - N), (0, 0)))
    src = jnp.concatenate([edge_index[0],
                           jnp.zeros((EP - E,), jnp.int32)])
    dst = jnp.concatenate([edge_index[1],
                           jnp.full((EP - E,), N, jnp.int32)])

    eye4 = jnp.eye(4, 16, dtype=F32)
    wl1 = (g1_al[:, :, None] * eye4[:, None, :]).reshape(HID, 16)
    wr1 = (g1_ar[:, :, None] * eye4[:, None, :]).reshape(HID, 16)
    eye1 = jnp.eye(1, 16, dtype=F32)
    wl2 = (g2_al[:, :, None] * eye1[:, None, :]).reshape(HID, 16)
    wr2 = (g2_ar[:, :, None] * eye1[:, None, :]).reshape(HID, 16)

    bh1 = jnp.concatenate(
        [jnp.repeat(jnp.eye(4, dtype=F32), 32, axis=1),
         jnp.zeros((12, HID), F32)], axis=0)
    bh2 = jnp.concatenate(
        [jnp.ones((1, HID), F32), jnp.zeros((15, HID), F32)], axis=0)

    r2 = lambda v: v.reshape(1, -1)

    f, g0, g1t = _tc1(xp, fa_w1, r2(fa_b1), fa_w2, r2(fa_b2))

    s0, s1 = _sc_moments(src, dst, g0, g1t)

    t1, r1 = _tc2(s0, s1, f, mp_w, r2(mp_b), g1_w, wl1, wr1)

    p0, p1 = _sc_gat(4, src, dst, t1, r1)

    x1, t2, r2t = _tc3(p0, p1, bh1, r2(g1_b.reshape(-1)), g2_w, wl2, wr2)

    q0, q1 = _sc_gat(1, src, dst, t2, r2t)

    logits = _tc4(q0, q1, x1, bh2, r2(g2_b), agg_w[:HID], agg_w[HID:],
                  r2(agg_b), c_w1, r2(c_b1), c_w2, r2(c_b2))

    return logits[:N]


# speedup vs baseline: 19.0063x; 1.4527x over previous
import functools

import jax
import jax.numpy as jnp
from jax import lax
from jax.experimental import pallas as pl
from jax.experimental.pallas import tpu as pltpu
from jax.experimental.pallas import tpu_sc as plsc

N = 10000
NP = 10240
AR = 10016
E = 320000
EP = 327680
EPA = EP + 256
CHM = 128
CH = 64
D = 128
HID = 128
OUT = 64
MW = 104
TW = 144
NC, NS = 2, 16

_BLK = 1024
_GRID = NP // _BLK

F32 = jnp.float32


def _dot(a, b):
    return jax.lax.dot_general(a, b, (((1,), (0,)), ((), ())),
                               precision=jax.lax.Precision.HIGHEST,
                               preferred_element_type=F32)


def _tc1_body(x_ref, w1_ref, b1_ref, w2_ref, b2_ref,
              f_ref, ta_ref, tb_ref, tc_ref, td_ref):
    x = x_ref[...]
    hgate = jnp.maximum(_dot(x, w1_ref[...]) + b1_ref[...], 0.0)
    gates = jax.nn.sigmoid(_dot(hgate, w2_ref[...]) + b2_ref[...])
    f = x * gates
    f2 = f * f
    f3 = f2 * f
    ones1 = jnp.ones((x.shape[0], 1), F32)
    z31 = jnp.zeros((x.shape[0], 31), F32)
    f_ref[...] = f
    ta_ref[...] = f[:, :104]
    tb_ref[...] = jnp.concatenate([f[:, 104:], f2[:, :80]], axis=1)
    tc_ref[...] = jnp.concatenate([f2[:, 80:], f3[:, :56]], axis=1)
    td_ref[...] = jnp.concatenate([f3[:, 56:], ones1, z31], axis=1)


def _tc1(x, fa_w1, fa_b1, fa_w2, fa_b2):
    row = lambda i: (i, 0)
    full = lambda i: (0, 0)
    mom = pl.BlockSpec((_BLK, MW), row)
    return pl.pallas_call(
        _tc1_body,
        grid=(_GRID,),
        in_specs=[
            pl.BlockSpec((_BLK, D), row),
            pl.BlockSpec((D, 32), full),
            pl.BlockSpec((1, 32), full),
            pl.BlockSpec((32, D), full),
            pl.BlockSpec((1, D), full),
        ],
        out_specs=[pl.BlockSpec((_BLK, D), row), mom, mom, mom, mom],
        out_shape=[
            jax.ShapeDtypeStruct((NP, D), F32),
            jax.ShapeDtypeStruct((NP, MW), F32),
            jax.ShapeDtypeStruct((NP, MW), F32),
            jax.ShapeDtypeStruct((NP, MW), F32),
            jax.ShapeDtypeStruct((NP, MW), F32),
        ],
    )(x, fa_w1, fa_b1, fa_w2, fa_b2)


def _tc2_body(sa_ref, sb_ref, sc_ref, sd_ref, f_ref, mpw_ref, mpb_ref,
              gw_ref, wl_ref, wr_ref, t_ref, r_ref):
    sa = sa_ref[...]
    sb = sb_ref[...]
    sc = sc_ref[...]
    sd = sd_ref[...]
    f = f_ref[...]
    cnt = sd[:, 72:73]
    d = jnp.maximum(cnt, 1.0)
    m1 = jnp.concatenate([sa, sb[:, :24]], axis=1) / d
    m2 = jnp.concatenate([sb[:, 24:], sc[:, :48]], axis=1) / d
    m3 = jnp.concatenate([sc[:, 48:], sd[:, :72]], axis=1) / d
    var = jnp.maximum(m2 - m1 * m1, 0.0)
    t = var + 1e-6
    skew = (m3 - 3.0 * m1 * m2 + 2.0 * m1 * m1 * m1) / (t * jnp.sqrt(t) + 1e-6)
    mixed = jnp.concatenate([f, m1, var, skew], axis=1)
    h = jnp.maximum(_dot(mixed, mpw_ref[...]) + mpb_ref[...], 0.0)
    hp = _dot(h, gw_ref[...])
    el16 = _dot(hp, wl_ref[...])
    er16 = _dot(hp, wr_ref[...])
    t_ref[...] = jnp.concatenate([hp, el16], axis=1)
    r_ref[...] = er16


def _tc2(sa, sb, sc, sd, f, mp_w, mp_b, g1_w, wl1, wr1):
    row = lambda i: (i, 0)
    full = lambda i: (0, 0)
    mom = pl.BlockSpec((_BLK, MW), row)
    return pl.pallas_call(
        _tc2_body,
        grid=(_GRID,),
        in_specs=[
            mom, mom, mom, mom,
            pl.BlockSpec((_BLK, D), row),
            pl.BlockSpec((4 * D, HID), full),
            pl.BlockSpec((1, HID), full),
            pl.BlockSpec((HID, HID), full),
            pl.BlockSpec((HID, 16), full),
            pl.BlockSpec((HID, 16), full),
        ],
        out_specs=[
            pl.BlockSpec((_BLK, TW), row),
            pl.BlockSpec((_BLK, 16), row),
        ],
        out_shape=[
            jax.ShapeDtypeStruct((NP, TW), F32),
            jax.ShapeDtypeStruct((NP, 16), F32),
        ],
    )(sa, sb, sc, sd, f, mp_w, mp_b, g1_w, wl1, wr1)


def _tc3_body(p0_ref, p1_ref, bh_ref, g1b_ref, gw_ref, wl_ref, wr_ref,
              x1_ref, t_ref, r_ref):
    p = p0_ref[...] + p1_ref[...]
    den = _dot(p[:, :16], bh_ref[...])
    x1 = p[:, 16:] / (den + 1e-9) + g1b_ref[...]
    x1 = jnp.where(x1 > 0, x1, jnp.exp(x1) - 1.0)
    hp = _dot(x1, gw_ref[...])
    el16 = _dot(hp, wl_ref[...])
    er16 = _dot(hp, wr_ref[...])
    x1_ref[...] = x1
    t_ref[...] = jnp.concatenate([hp, el16], axis=1)
    r_ref[...] = er16


def _tc3(p0, p1, bh1, g1_b, g2_w, wl2, wr2):
    row = lambda i: (i, 0)
    full = lambda i: (0, 0)
    return pl.pallas_call(
        _tc3_body,
        grid=(_GRID,),
        in_specs=[
            pl.BlockSpec((_BLK, TW), row),
            pl.BlockSpec((_BLK, TW), row),
            pl.BlockSpec((16, HID), full),
            pl.BlockSpec((1, HID), full),
            pl.BlockSpec((HID, HID), full),
            pl.BlockSpec((HID, 16), full),
            pl.BlockSpec((HID, 16), full),
        ],
        out_specs=[
            pl.BlockSpec((_BLK, HID), row),
            pl.BlockSpec((_BLK, TW), row),
            pl.BlockSpec((_BLK, 16), row),
        ],
        out_shape=[
            jax.ShapeDtypeStruct((NP, HID), F32),
            jax.ShapeDtypeStruct((NP, TW), F32),
            jax.ShapeDtypeStruct((NP, 16), F32),
        ],
    )(p0, p1, bh1, g1_b, g2_w, wl2, wr2)


def _tc4_body(q0_ref, q1_ref, x1_ref, bh_ref, g2b_ref, aw1_ref, aw2_ref,
              ab_ref, cw1_ref, cb1_ref, cw2_ref, cb2_ref, out_ref):
    q = q0_ref[...] + q1_ref[...]
    den = _dot(q[:, :16], bh_ref[...])
    x2 = q[:, 16:] / (den + 1e-9) + g2b_ref[...]
    x2 = jnp.where(x2 > 0, x2, jnp.exp(x2) - 1.0)
    x1 = x1_ref[...]
    agg = jnp.maximum(_dot(x1, aw1_ref[...]) + _dot(x2, aw2_ref[...])
                      + ab_ref[...], 0.0)
    hc = jnp.maximum(_dot(agg, cw1_ref[...]) + cb1_ref[...], 0.0)
    out_ref[...] = _dot(hc, cw2_ref[...]) + cb2_ref[...]


def _tc4(q0, q1, x1, bh2, g2_b, aw1, aw2, agg_b, c_w1, c_b1, c_w2, c_b2):
    row = lambda i: (i, 0)
    full = lambda i: (0, 0)
    return pl.pallas_call(
        _tc4_body,
        grid=(_GRID,),
        in_specs=[
            pl.BlockSpec((_BLK, TW), row),
            pl.BlockSpec((_BLK, TW), row),
            pl.BlockSpec((_BLK, HID), row),
            pl.BlockSpec((16, HID), full),
            pl.BlockSpec((1, HID), full),
            pl.BlockSpec((HID, HID), full),
            pl.BlockSpec((HID, HID), full),
            pl.BlockSpec((1, HID), full),
            pl.BlockSpec((HID, HID), full),
            pl.BlockSpec((1, HID), full),
            pl.BlockSpec((HID, OUT), full),
            pl.BlockSpec((1, OUT), full),
        ],
        out_specs=[pl.BlockSpec((_BLK, OUT), row)],
        out_shape=[jax.ShapeDtypeStruct((NP, OUT), F32)],
    )(q0, q1, x1, bh2, g2_b, aw1, aw2, agg_b, c_w1, c_b1, c_w2, c_b2)[0]


def _zero_vmem(buf, rows, cols):
    z = jnp.zeros((16,), F32)
    offs = [j * 16 for j in range(cols // 16)]
    if cols % 16:
        offs.append(cols - 16)

    def row_body(r, _):
        for o in offs:
            buf[r, pl.ds(o, 16)] = z
        return 0

    lax.fori_loop(0, rows, row_body, 0)


def _zero_acc(acc, zbuf, wid, zr):
    nchk = AR // zr
    rem = AR - nchk * zr
    for i in range(-(-nchk // NS)):
        cidx = wid + i * NS

        @pl.when(cidx < nchk)
        def _():
            pltpu.sync_copy(zbuf, acc.at[pl.ds(cidx * zr, zr), :])

    if rem:
        @pl.when(wid == 0)
        def _():
            pltpu.sync_copy(zbuf.at[pl.ds(0, rem), :],
                            acc.at[pl.ds(nchk * zr, rem), :])


def _copy_out(acc, out_ref, wid, zr):
    nchk = AR // zr
    rem = AR - nchk * zr
    for i in range(-(-nchk // NS)):
        cidx = wid + i * NS

        @pl.when(cidx < nchk)
        def _():
            pltpu.sync_copy(acc.at[pl.ds(cidx * zr, zr), :],
                            out_ref.at[pl.ds(cidx * zr, zr), :])

    if rem:
        @pl.when(wid == 0)
        def _():
            pltpu.sync_copy(acc.at[pl.ds(nchk * zr, rem), :],
                            out_ref.at[pl.ds(nchk * zr, rem), :])


def _moment_kernel(srcp, dstp, ta, tb, tc, td, sa, sb, sc, sd,
                   idx_s0, idx_s1, idx_d0, idx_d1, rows0, rows1, acc,
                   si0, si1, sg0, sg1):
    cid = lax.axis_index("c")
    wid = lax.axis_index("s")
    per_sub = EP // NS
    nch = per_sub // CHM
    base = wid * per_sub

    def idx_wait(sem, dst):
        pltpu.make_async_copy(srcp.at[pl.ds(0, CHM)], dst, sem).wait()

    def one_pass(t0, t1, o0, o1):
        _zero_vmem(rows0, CHM, MW)
        _zero_acc(acc, rows0, wid, CHM)
        plsc.subcore_barrier()

        pltpu.async_copy(srcp.at[pl.ds(base, CHM)], idx_s0, si0)
        pltpu.async_copy(dstp.at[pl.ds(base, CHM)], idx_d0, si0)
        pltpu.async_copy(srcp.at[pl.ds(base + CHM, CHM)], idx_s1, si1)
        pltpu.async_copy(dstp.at[pl.ds(base + CHM, CHM)], idx_d1, si1)

        def body(k, _):
            off0 = base + 2 * k * CHM
            idx_wait(si0, idx_s0)
            idx_wait(si0, idx_d0)

            @pl.when(cid == 0)
            def _():
                pltpu.async_copy(t0.at[idx_s0], rows0, sg0)

            @pl.when(cid == 1)
            def _():
                pltpu.async_copy(t1.at[idx_s0], rows0, sg0)

            idx_wait(si1, idx_s1)
            idx_wait(si1, idx_d1)

            @pl.when(cid == 0)
            def _():
                pltpu.async_copy(t0.at[idx_s1], rows1, sg1)

            @pl.when(cid == 1)
            def _():
                pltpu.async_copy(t1.at[idx_s1], rows1, sg1)

            pltpu.make_async_copy(t0.at[idx_s0], rows0, sg0).wait()
            pltpu.sync_copy(rows0, acc.at[idx_d0], add=True)
            pltpu.async_copy(srcp.at[pl.ds(off0 + 2 * CHM, CHM)], idx_s0, si0)
            pltpu.async_copy(dstp.at[pl.ds(off0 + 2 * CHM, CHM)], idx_d0, si0)

            pltpu.make_async_copy(t0.at[idx_s1], rows1, sg1).wait()
            pltpu.sync_copy(rows1, acc.at[idx_d1], add=True)
            pltpu.async_copy(srcp.at[pl.ds(off0 + 3 * CHM, CHM)], idx_s1, si1)
            pltpu.async_copy(dstp.at[pl.ds(off0 + 3 * CHM, CHM)], idx_d1, si1)
            return 0

        lax.fori_loop(0, nch // 2, body, 0)

        idx_wait(si0, idx_s0)
        idx_wait(si0, idx_d0)
        idx_wait(si1, idx_s1)
        idx_wait(si1, idx_d1)
        plsc.subcore_barrier()

        @pl.when(cid == 0)
        def _():
            _copy_out(acc, o0, wid, CHM)

        @pl.when(cid == 1)
        def _():
            _copy_out(acc, o1, wid, CHM)

        plsc.subcore_barrier()

    one_pass(ta, tc, sa, sc)
    one_pass(tb, td, sb, sd)


def _sc_moments(srcp, dstp, ta, tb, tc, td):
    mesh = plsc.VectorSubcoreMesh(core_axis_name="c", subcore_axis_name="s")
    out = jax.ShapeDtypeStruct((NP, MW), F32)
    kfn = pl.kernel(
        _moment_kernel,
        out_type=[out, out, out, out],
        mesh=mesh,
        compiler_params=pltpu.CompilerParams(use_tc_tiling_on_sc=False),
        scratch_types=[
            pltpu.VMEM((CHM,), jnp.int32),
            pltpu.VMEM((CHM,), jnp.int32),
            pltpu.VMEM((CHM,), jnp.int32),
            pltpu.VMEM((CHM,), jnp.int32),
            pltpu.VMEM((CHM, MW), F32),
            pltpu.VMEM((CHM, MW), F32),
            pltpu.VMEM_SHARED((AR, MW), F32),
            pltpu.SemaphoreType.DMA,
            pltpu.SemaphoreType.DMA,
            pltpu.SemaphoreType.DMA,
            pltpu.SemaphoreType.DMA,
        ],
    )
    return kfn(srcp, dstp, ta, tb, tc, td)


def _gat_kernel(heads, srcp, dstp, tbl, rtbl, p0_out, p1_out,
                idx_s0, idx_s1, idx_d0, idx_d1, sidx0, sidx1,
                rbuf0, rbuf1, ebuf0, ebuf1, obuf0, obuf1, acc,
                si0, si1, sg0, sg1, ss0, ss1):
    cid = lax.axis_index("c")
    wid = lax.axis_index("s")

    _zero_vmem(obuf0, CH, TW)
    _zero_acc(acc, obuf0, wid, CH)
    plsc.subcore_barrier()

    w32 = wid * NC + cid
    per_w = EP // (NC * NS)
    nch = per_w // CH
    base = w32 * per_w
    bph = (128 // heads) // 16

    def idx_wait(sem, dst):
        pltpu.make_async_copy(srcp.at[pl.ds(0, CH)], dst, sem).wait()

    def compute(rbuf, ebuf, obuf):
        def edge(e, _):
            vel = rbuf[e, pl.ds(128, 16)]
            ver = ebuf[e, pl.ds(0, 16)]
            s = vel + ver
            w = jnp.exp(jnp.where(s > 0, s, 0.2 * s))
            obuf[e, pl.ds(0, 16)] = w
            for h in range(heads):
                wh = w[h]
                for b in range(bph):
                    c0 = h * (128 // heads) + b * 16
                    obuf[e, pl.ds(16 + c0, 16)] = rbuf[e, pl.ds(c0, 16)] * wh
            return 0

        lax.fori_loop(0, CH, edge, 0)

    def half(k, off, idx_s, idx_d, sidx, rbuf, ebuf, obuf, si, sg, ss):
        pltpu.make_async_copy(tbl.at[idx_s], rbuf, sg).wait()
        pltpu.make_async_copy(rtbl.at[idx_d], ebuf, sg).wait()

        @pl.when(k > 0)
        def _():
            pltpu.make_async_copy(obuf, acc.at[sidx], ss).wait()

        for j in range(CH // 16):
            sidx[pl.ds(j * 16, 16)] = idx_d[pl.ds(j * 16, 16)]
        pltpu.async_copy(srcp.at[pl.ds(off + 2 * CH, CH)], idx_s, si)
        pltpu.async_copy(dstp.at[pl.ds(off + 2 * CH, CH)], idx_d, si)

        compute(rbuf, ebuf, obuf)
        pltpu.async_copy(obuf, acc.at[sidx], ss, add=True)

    pltpu.async_copy(srcp.at[pl.ds(base, CH)], idx_s0, si0)
    pltpu.async_copy(dstp.at[pl.ds(base, CH)], idx_d0, si0)
    pltpu.async_copy(srcp.at[pl.ds(base + CH, CH)], idx_s1, si1)
    pltpu.async_copy(dstp.at[pl.ds(base + CH, CH)], idx_d1, si1)

    def body(k, _):
        off0 = base + 2 * k * CH
        idx_wait(si0, idx_s0)
        idx_wait(si0, idx_d0)
        pltpu.async_copy(tbl.at[idx_s0], rbuf0, sg0)
        pltpu.async_copy(rtbl.at[idx_d0], ebuf0, sg0)
        idx_wait(si1, idx_s1)
        idx_wait(si1, idx_d1)
        pltpu.async_copy(tbl.at[idx_s1], rbuf1, sg1)
        pltpu.async_copy(rtbl.at[idx_d1], ebuf1, sg1)

        half(k, off0, idx_s0, idx_d0, sidx0, rbuf0, ebuf0, obuf0,
             si0, sg0, ss0)
        half(k, off0 + CH, idx_s1, idx_d1, sidx1, rbuf1, ebuf1, obuf1,
             si1, sg1, ss1)
        return 0

    lax.fori_loop(0, nch // 2, body, 0)

    pltpu.make_async_copy(obuf0, acc.at[sidx0], ss0).wait()
    pltpu.make_async_copy(obuf1, acc.at[sidx1], ss1).wait()
    idx_wait(si0, idx_s0)
    idx_wait(si0, idx_d0)
    idx_wait(si1, idx_s1)
    idx_wait(si1, idx_d1)
    plsc.subcore_barrier()

    @pl.when(cid == 0)
    def _():
        _copy_out(acc, p0_out, wid, CH)

    @pl.when(cid == 1)
    def _():
        _copy_out(acc, p1_out, wid, CH)


def _sc_gat(heads, srcp, dstp, tbl, rtbl):
    mesh = plsc.VectorSubcoreMesh(core_axis_name="c", subcore_axis_name="s")
    kfn = pl.kernel(
        functools.partial(_gat_kernel, heads),
        out_type=[
            jax.ShapeDtypeStruct((NP, TW), F32),
            jax.ShapeDtypeStruct((NP, TW), F32),
        ],
        mesh=mesh,
        compiler_params=pltpu.CompilerParams(use_tc_tiling_on_sc=False),
        scratch_types=[
            pltpu.VMEM((CH,), jnp.int32),
            pltpu.VMEM((CH,), jnp.int32),
            pltpu.VMEM((CH,), jnp.int32),
            pltpu.VMEM((CH,), jnp.int32),
            pltpu.VMEM((CH,), jnp.int32),
            pltpu.VMEM((CH,), jnp.int32),
            pltpu.VMEM((CH, TW), F32),
            pltpu.VMEM((CH, TW), F32),
            pltpu.VMEM((CH, 16), F32),
            pltpu.VMEM((CH, 16), F32),
            pltpu.VMEM((CH, TW), F32),
            pltpu.VMEM((CH, TW), F32),
            pltpu.VMEM_SHARED((AR, TW), F32),
            pltpu.SemaphoreType.DMA,
            pltpu.SemaphoreType.DMA,
            pltpu.SemaphoreType.DMA,
            pltpu.SemaphoreType.DMA,
            pltpu.SemaphoreType.DMA,
            pltpu.SemaphoreType.DMA,
        ],
    )
    return kfn(srcp, dstp, tbl, rtbl)


def kernel(x, edge_index, fa_w1, fa_b1, fa_w2, fa_b2, mp_w, mp_b,
           g1_w, g1_al, g1_ar, g1_b, g2_w, g2_al, g2_ar, g2_b,
           agg_w, agg_b, c_w1, c_b1, c_w2, c_b2):
    xp = jnp.pad(x, ((0, NP - N), (0, 0)))
    src = jnp.concatenate([edge_index[0],
                           jnp.zeros((EPA - E,), jnp.int32)])
    dst = jnp.concatenate([edge_index[1],
                           jnp.full((EP - E,), N, jnp.int32),
                           jnp.zeros((EPA - EP,), jnp.int32)])

    eye4 = jnp.eye(4, 16, dtype=F32)
    wl1 = (g1_al[:, :, None] * eye4[:, None, :]).reshape(HID, 16)
    wr1 = (g1_ar[:, :, None] * eye4[:, None, :]).reshape(HID, 16)
    eye1 = jnp.eye(1, 16, dtype=F32)
    wl2 = (g2_al[:, :, None] * eye1[:, None, :]).reshape(HID, 16)
    wr2 = (g2_ar[:, :, None] * eye1[:, None, :]).reshape(HID, 16)

    bh1 = jnp.concatenate(
        [jnp.repeat(jnp.eye(4, dtype=F32), 32, axis=1),
         jnp.zeros((12, HID), F32)], axis=0)
    bh2 = jnp.concatenate(
        [jnp.ones((1, HID), F32), jnp.zeros((15, HID), F32)], axis=0)

    r2 = lambda v: v.reshape(1, -1)

    f, ta, tb, tc, td = _tc1(xp, fa_w1, r2(fa_b1), fa_w2, r2(fa_b2))

    sa, sb, sc, sd = _sc_moments(src, dst, ta, tb, tc, td)

    t1, r1 = _tc2(sa, sb, sc, sd, f, mp_w, r2(mp_b), g1_w, wl1, wr1)

    p0, p1 = _sc_gat(4, src, dst, t1, r1)

    x1, t2, r2t = _tc3(p0, p1, bh1, r2(g1_b), g2_w, wl2, wr2)

    q0, q1 = _sc_gat(1, src, dst, t2, r2t)

    logits = _tc4(q0, q1, x1, bh2, r2(g2_b), agg_w[:HID], agg_w[HID:],
                  r2(agg_b), c_w1, r2(c_b1), c_w2, r2(c_b2))

    return logits[:N]


# speedup vs baseline: 22.8737x; 1.2035x over previous
import functools

import jax
import jax.numpy as jnp
from jax import lax
from jax.experimental import pallas as pl
from jax.experimental.pallas import tpu as pltpu
from jax.experimental.pallas import tpu_sc as plsc

N = 10000
NP = 10240
AR = 10016
E = 320000
EP = 327680
EPA = EP + 256
CHM = 128
CH = 64
D = 128
HID = 128
OUT = 64
MW = 104
TW = 144
NC, NS = 2, 16

_BLK = 1024
_GRID = NP // _BLK

F32 = jnp.float32


def _dot(a, b):
    return jax.lax.dot_general(a, b, (((1,), (0,)), ((), ())),
                               precision=jax.lax.Precision.HIGHEST,
                               preferred_element_type=F32)


def _tc1_body(x_ref, w1_ref, b1_ref, w2_ref, b2_ref,
              f_ref, ta_ref, tb_ref, tc_ref, td_ref):
    x = x_ref[...]
    hgate = jnp.maximum(_dot(x, w1_ref[...]) + b1_ref[...], 0.0)
    gates = jax.nn.sigmoid(_dot(hgate, w2_ref[...]) + b2_ref[...])
    f = x * gates
    f2 = f * f
    f3 = f2 * f
    ones1 = jnp.ones((x.shape[0], 1), F32)
    z31 = jnp.zeros((x.shape[0], 31), F32)
    f_ref[...] = f
    ta_ref[...] = f[:, :104]
    tb_ref[...] = jnp.concatenate([f[:, 104:], f2[:, :80]], axis=1)
    tc_ref[...] = jnp.concatenate([f2[:, 80:], f3[:, :56]], axis=1)
    td_ref[...] = jnp.concatenate([f3[:, 56:], ones1, z31], axis=1)


def _tc1(x, fa_w1, fa_b1, fa_w2, fa_b2):
    row = lambda i: (i, 0)
    full = lambda i: (0, 0)
    mom = pl.BlockSpec((_BLK, MW), row)
    return pl.pallas_call(
        _tc1_body,
        grid=(_GRID,),
        in_specs=[
            pl.BlockSpec((_BLK, D), row),
            pl.BlockSpec((D, 32), full),
            pl.BlockSpec((1, 32), full),
            pl.BlockSpec((32, D), full),
            pl.BlockSpec((1, D), full),
        ],
        out_specs=[pl.BlockSpec((_BLK, D), row), mom, mom, mom, mom],
        out_shape=[
            jax.ShapeDtypeStruct((NP, D), F32),
            jax.ShapeDtypeStruct((NP, MW), F32),
            jax.ShapeDtypeStruct((NP, MW), F32),
            jax.ShapeDtypeStruct((NP, MW), F32),
            jax.ShapeDtypeStruct((NP, MW), F32),
        ],
    )(x, fa_w1, fa_b1, fa_w2, fa_b2)


def _tc2_body(sa_ref, sb_ref, sc_ref, sd_ref, f_ref, mpw_ref, mpb_ref,
              gw_ref, wl_ref, wr_ref, t_ref, r_ref):
    sa = sa_ref[...]
    sb = sb_ref[...]
    sc = sc_ref[...]
    sd = sd_ref[...]
    f = f_ref[...]
    cnt = sd[:, 72:73]
    d = jnp.maximum(cnt, 1.0)
    m1 = jnp.concatenate([sa, sb[:, :24]], axis=1) / d
    m2 = jnp.concatenate([sb[:, 24:], sc[:, :48]], axis=1) / d
    m3 = jnp.concatenate([sc[:, 48:], sd[:, :72]], axis=1) / d
    var = jnp.maximum(m2 - m1 * m1, 0.0)
    t = var + 1e-6
    skew = (m3 - 3.0 * m1 * m2 + 2.0 * m1 * m1 * m1) / (t * jnp.sqrt(t) + 1e-6)
    mixed = jnp.concatenate([f, m1, var, skew], axis=1)
    h = jnp.maximum(_dot(mixed, mpw_ref[...]) + mpb_ref[...], 0.0)
    hp = _dot(h, gw_ref[...])
    el16 = _dot(hp, wl_ref[...])
    er16 = _dot(hp, wr_ref[...])
    t_ref[...] = jnp.concatenate([hp, el16], axis=1)
    r_ref[...] = er16


def _tc2(sa, sb, sc, sd, f, mp_w, mp_b, g1_w, wl1, wr1):
    row = lambda i: (i, 0)
    full = lambda i: (0, 0)
    mom = pl.BlockSpec((_BLK, MW), row)
    return pl.pallas_call(
        _tc2_body,
        grid=(_GRID,),
        in_specs=[
            mom, mom, mom, mom,
            pl.BlockSpec((_BLK, D), row),
            pl.BlockSpec((4 * D, HID), full),
            pl.BlockSpec((1, HID), full),
            pl.BlockSpec((HID, HID), full),
            pl.BlockSpec((HID, 16), full),
            pl.BlockSpec((HID, 16), full),
        ],
        out_specs=[
            pl.BlockSpec((_BLK, TW), row),
            pl.BlockSpec((_BLK, 16), row),
        ],
        out_shape=[
            jax.ShapeDtypeStruct((NP, TW), F32),
            jax.ShapeDtypeStruct((NP, 16), F32),
        ],
    )(sa, sb, sc, sd, f, mp_w, mp_b, g1_w, wl1, wr1)


def _tc3_body(p0_ref, p1_ref, bh_ref, g1b_ref, gw_ref, wl_ref, wr_ref,
              x1_ref, t_ref, r_ref):
    p = p0_ref[...] + p1_ref[...]
    den = _dot(p[:, :16], bh_ref[...])
    x1 = p[:, 16:] / (den + 1e-9) + g1b_ref[...]
    x1 = jnp.where(x1 > 0, x1, jnp.exp(x1) - 1.0)
    hp = _dot(x1, gw_ref[...])
    el16 = _dot(hp, wl_ref[...])
    er16 = _dot(hp, wr_ref[...])
    x1_ref[...] = x1
    t_ref[...] = jnp.concatenate([hp, el16], axis=1)
    r_ref[...] = er16


def _tc3(p0, p1, bh1, g1_b, g2_w, wl2, wr2):
    row = lambda i: (i, 0)
    full = lambda i: (0, 0)
    return pl.pallas_call(
        _tc3_body,
        grid=(_GRID,),
        in_specs=[
            pl.BlockSpec((_BLK, TW), row),
            pl.BlockSpec((_BLK, TW), row),
            pl.BlockSpec((16, HID), full),
            pl.BlockSpec((1, HID), full),
            pl.BlockSpec((HID, HID), full),
            pl.BlockSpec((HID, 16), full),
            pl.BlockSpec((HID, 16), full),
        ],
        out_specs=[
            pl.BlockSpec((_BLK, HID), row),
            pl.BlockSpec((_BLK, TW), row),
            pl.BlockSpec((_BLK, 16), row),
        ],
        out_shape=[
            jax.ShapeDtypeStruct((NP, HID), F32),
            jax.ShapeDtypeStruct((NP, TW), F32),
            jax.ShapeDtypeStruct((NP, 16), F32),
        ],
    )(p0, p1, bh1, g1_b, g2_w, wl2, wr2)


def _tc4_body(q0_ref, q1_ref, x1_ref, bh_ref, g2b_ref, aw1_ref, aw2_ref,
              ab_ref, cw1_ref, cb1_ref, cw2_ref, cb2_ref, out_ref):
    q = q0_ref[...] + q1_ref[...]
    den = _dot(q[:, :16], bh_ref[...])
    x2 = q[:, 16:] / (den + 1e-9) + g2b_ref[...]
    x2 = jnp.where(x2 > 0, x2, jnp.exp(x2) - 1.0)
    x1 = x1_ref[...]
    agg = jnp.maximum(_dot(x1, aw1_ref[...]) + _dot(x2, aw2_ref[...])
                      + ab_ref[...], 0.0)
    hc = jnp.maximum(_dot(agg, cw1_ref[...]) + cb1_ref[...], 0.0)
    out_ref[...] = _dot(hc, cw2_ref[...]) + cb2_ref[...]


def _tc4(q0, q1, x1, bh2, g2_b, aw1, aw2, agg_b, c_w1, c_b1, c_w2, c_b2):
    row = lambda i: (i, 0)
    full = lambda i: (0, 0)
    return pl.pallas_call(
        _tc4_body,
        grid=(_GRID,),
        in_specs=[
            pl.BlockSpec((_BLK, TW), row),
            pl.BlockSpec((_BLK, TW), row),
            pl.BlockSpec((_BLK, HID), row),
            pl.BlockSpec((16, HID), full),
            pl.BlockSpec((1, HID), full),
            pl.BlockSpec((HID, HID), full),
            pl.BlockSpec((HID, HID), full),
            pl.BlockSpec((1, HID), full),
            pl.BlockSpec((HID, HID), full),
            pl.BlockSpec((1, HID), full),
            pl.BlockSpec((HID, OUT), full),
            pl.BlockSpec((1, OUT), full),
        ],
        out_specs=[pl.BlockSpec((_BLK, OUT), row)],
        out_shape=[jax.ShapeDtypeStruct((NP, OUT), F32)],
    )(q0, q1, x1, bh2, g2_b, aw1, aw2, agg_b, c_w1, c_b1, c_w2, c_b2)[0]


def _zero_vmem(buf, rows, cols):
    z = jnp.zeros((16,), F32)
    offs = [j * 16 for j in range(cols // 16)]
    if cols % 16:
        offs.append(cols - 16)

    def row_body(r, _):
        for o in offs:
            buf[r, pl.ds(o, 16)] = z
        return 0

    lax.fori_loop(0, rows, row_body, 0)


def _zero_acc(acc, zbuf, wid, zr):
    nchk = AR // zr
    rem = AR - nchk * zr
    for i in range(-(-nchk // NS)):
        cidx = wid + i * NS

        @pl.when(cidx < nchk)
        def _():
            pltpu.sync_copy(zbuf, acc.at[pl.ds(cidx * zr, zr), :])

    if rem:
        @pl.when(wid == 0)
        def _():
            pltpu.sync_copy(zbuf.at[pl.ds(0, rem), :],
                            acc.at[pl.ds(nchk * zr, rem), :])


def _copy_out(acc, out_ref, wid, zr):
    nchk = AR // zr
    rem = AR - nchk * zr
    for i in range(-(-nchk // NS)):
        cidx = wid + i * NS

        @pl.when(cidx < nchk)
        def _():
            pltpu.sync_copy(acc.at[pl.ds(cidx * zr, zr), :],
                            out_ref.at[pl.ds(cidx * zr, zr), :])

    if rem:
        @pl.when(wid == 0)
        def _():
            pltpu.sync_copy(acc.at[pl.ds(nchk * zr, rem), :],
                            out_ref.at[pl.ds(nchk * zr, rem), :])


def _copy_idx(dst, src):
    for j in range(src.shape[0] // 16):
        dst[pl.ds(j * 16, 16)] = src[pl.ds(j * 16, 16)]


def _moment_kernel(srcp, dstp, ta, tb, tc, td, sa, sb, sc, sd,
                   idx_s0, idx_s1, idx_d0, idx_d1, sidx0, sidx1,
                   rows0, rows1, acc, si0, si1, sg0, sg1, ss0, ss1):
    cid = lax.axis_index("c")
    wid = lax.axis_index("s")
    per_sub = EP // NS
    nch = per_sub // CHM
    base = wid * per_sub

    idx_bufs = ((idx_s0, idx_d0, sidx0, rows0, si0, sg0, ss0),
                (idx_s1, idx_d1, sidx1, rows1, si1, sg1, ss1))

    def idx_wait(sem, dst):
        pltpu.make_async_copy(srcp.at[pl.ds(0, CHM)], dst, sem).wait()

    def one_pass(t0, t1, o0, o1):
        _zero_vmem(rows0, CHM, MW)
        _zero_acc(acc, rows0, wid, CHM)
        plsc.subcore_barrier()

        def gather(idx_s, rows, sg):
            @pl.when(cid == 0)
            def _():
                pltpu.async_copy(t0.at[idx_s], rows, sg)

            @pl.when(cid == 1)
            def _():
                pltpu.async_copy(t1.at[idx_s], rows, sg)

        pltpu.async_copy(srcp.at[pl.ds(base, CHM)], idx_s0, si0)
        pltpu.async_copy(dstp.at[pl.ds(base, CHM)], idx_d0, si0)
        pltpu.async_copy(srcp.at[pl.ds(base + CHM, CHM)], idx_s1, si1)
        pltpu.async_copy(dstp.at[pl.ds(base + CHM, CHM)], idx_d1, si1)
        idx_wait(si0, idx_s0)
        idx_wait(si0, idx_d0)
        gather(idx_s0, rows0, sg0)

        def half(k, c, b):
            idx_s, idx_d, sidx, rows, si, sg, ss = idx_bufs[b]
            nidx_s, nidx_d, nsidx, nrows, nsi, nsg, nss = idx_bufs[1 - b]
            @pl.when(c > 0)
            def _():
                pltpu.make_async_copy(nrows, acc.at[nsidx], nss).wait()

            idx_wait(nsi, nidx_s)
            idx_wait(nsi, nidx_d)
            gather(nidx_s, nrows, nsg)

            pltpu.make_async_copy(t0.at[idx_s], rows, sg).wait()
            _copy_idx(sidx, idx_d)
            off2 = base + (c + 2) * CHM
            pltpu.async_copy(srcp.at[pl.ds(off2, CHM)], idx_s, si)
            pltpu.async_copy(dstp.at[pl.ds(off2, CHM)], idx_d, si)

            pltpu.async_copy(rows, acc.at[sidx], ss, add=True)

        def body(k, _):
            half(k, 2 * k, 0)
            half(k, 2 * k + 1, 1)
            return 0

        lax.fori_loop(0, nch // 2, body, 0)

        pltpu.make_async_copy(t0.at[idx_s0], rows0, sg0).wait()
        pltpu.make_async_copy(rows1, acc.at[sidx1], ss1).wait()
        idx_wait(si1, idx_s1)
        idx_wait(si1, idx_d1)
        plsc.subcore_barrier()

        @pl.when(cid == 0)
        def _():
            _copy_out(acc, o0, wid, CHM)

        @pl.when(cid == 1)
        def _():
            _copy_out(acc, o1, wid, CHM)

        plsc.subcore_barrier()

    one_pass(ta, tc, sa, sc)
    one_pass(tb, td, sb, sd)


def _sc_moments(srcp, dstp, ta, tb, tc, td):
    mesh = plsc.VectorSubcoreMesh(core_axis_name="c", subcore_axis_name="s")
    out = jax.ShapeDtypeStruct((NP, MW), F32)
    kfn = pl.kernel(
        _moment_kernel,
        out_type=[out, out, out, out],
        mesh=mesh,
        compiler_params=pltpu.CompilerParams(use_tc_tiling_on_sc=False),
        scratch_types=[
            pltpu.VMEM((CHM,), jnp.int32),
            pltpu.VMEM((CHM,), jnp.int32),
            pltpu.VMEM((CHM,), jnp.int32),
            pltpu.VMEM((CHM,), jnp.int32),
            pltpu.VMEM((CHM,), jnp.int32),
            pltpu.VMEM((CHM,), jnp.int32),
            pltpu.VMEM((CHM, MW), F32),
            pltpu.VMEM((CHM, MW), F32),
            pltpu.VMEM_SHARED((AR, MW), F32),
            pltpu.SemaphoreType.DMA,
            pltpu.SemaphoreType.DMA,
            pltpu.SemaphoreType.DMA,
            pltpu.SemaphoreType.DMA,
            pltpu.SemaphoreType.DMA,
            pltpu.SemaphoreType.DMA,
        ],
    )
    return kfn(srcp, dstp, ta, tb, tc, td)


def _gat_kernel(heads, srcp, dstp, tbl, rtbl, p0_out, p1_out,
                idx_s0, idx_s1, idx_d0, idx_d1, sidx0, sidx1,
                rbuf0, rbuf1, ebuf0, ebuf1, obuf0, obuf1, acc,
                si0, si1, sg0, sg1, ss0, ss1):
    cid = lax.axis_index("c")
    wid = lax.axis_index("s")

    _zero_vmem(obuf0, CH, TW)
    _zero_acc(acc, obuf0, wid, CH)
    plsc.subcore_barrier()

    w32 = wid * NC + cid
    per_w = EP // (NC * NS)
    nch = per_w // CH
    base = w32 * per_w
    bph = (128 // heads) // 16

    def idx_wait(sem, dst):
        pltpu.make_async_copy(srcp.at[pl.ds(0, CH)], dst, sem).wait()

    def compute(rbuf, ebuf, obuf):
        def edge(e, _):
            vel = rbuf[e, pl.ds(128, 16)]
            ver = ebuf[e, pl.ds(0, 16)]
            s = vel + ver
            w = jnp.exp(jnp.where(s > 0, s, 0.2 * s))
            obuf[e, pl.ds(0, 16)] = w
            for h in range(heads):
                wh = w[h]
                for b in range(bph):
                    c0 = h * (128 // heads) + b * 16
                    obuf[e, pl.ds(16 + c0, 16)] = rbuf[e, pl.ds(c0, 16)] * wh
            return 0

        lax.fori_loop(0, CH, edge, 0)

    bufs = ((idx_s0, idx_d0, sidx0, rbuf0, ebuf0, obuf0, si0, sg0, ss0),
            (idx_s1, idx_d1, sidx1, rbuf1, ebuf1, obuf1, si1, sg1, ss1))

    def half(k, c, b):
        idx_s, idx_d, sidx, rbuf, ebuf, obuf, si, sg, ss = bufs[b]
        nidx_s, nidx_d, _, nrbuf, nebuf, _, nsi, nsg, _ = bufs[1 - b]
        idx_wait(nsi, nidx_s)
        idx_wait(nsi, nidx_d)
        pltpu.async_copy(tbl.at[nidx_s], nrbuf, nsg)
        pltpu.async_copy(rtbl.at[nidx_d], nebuf, nsg)

        pltpu.make_async_copy(tbl.at[idx_s], rbuf, sg).wait()
        pltpu.make_async_copy(rtbl.at[idx_d], ebuf, sg).wait()

        @pl.when(k > 0)
        def _():
            pltpu.make_async_copy(obuf, acc.at[sidx], ss).wait()

        _copy_idx(sidx, idx_d)
        off2 = base + (c + 2) * CH
        pltpu.async_copy(srcp.at[pl.ds(off2, CH)], idx_s, si)
        pltpu.async_copy(dstp.at[pl.ds(off2, CH)], idx_d, si)

        compute(rbuf, ebuf, obuf)
        pltpu.async_copy(obuf, acc.at[sidx], ss, add=True)

    pltpu.async_copy(srcp.at[pl.ds(base, CH)], idx_s0, si0)
    pltpu.async_copy(dstp.at[pl.ds(base, CH)], idx_d0, si0)
    pltpu.async_copy(srcp.at[pl.ds(base + CH, CH)], idx_s1, si1)
    pltpu.async_copy(dstp.at[pl.ds(base + CH, CH)], idx_d1, si1)
    idx_wait(si0, idx_s0)
    idx_wait(si0, idx_d0)
    pltpu.async_copy(tbl.at[idx_s0], rbuf0, sg0)
    pltpu.async_copy(rtbl.at[idx_d0], ebuf0, sg0)

    def body(k, _):
        half(k, 2 * k, 0)
        half(k, 2 * k + 1, 1)
        return 0

    lax.fori_loop(0, nch // 2, body, 0)

    pltpu.make_async_copy(tbl.at[idx_s0], rbuf0, sg0).wait()
    pltpu.make_async_copy(rtbl.at[idx_d0], ebuf0, sg0).wait()
    pltpu.make_async_copy(obuf0, acc.at[sidx0], ss0).wait()
    pltpu.make_async_copy(obuf1, acc.at[sidx1], ss1).wait()
    idx_wait(si1, idx_s1)
    idx_wait(si1, idx_d1)
    plsc.subcore_barrier()

    @pl.when(cid == 0)
    def _():
        _copy_out(acc, p0_out, wid, CH)

    @pl.when(cid == 1)
    def _():
        _copy_out(acc, p1_out, wid, CH)


def _sc_gat(heads, srcp, dstp, tbl, rtbl):
    mesh = plsc.VectorSubcoreMesh(core_axis_name="c", subcore_axis_name="s")
    kfn = pl.kernel(
        functools.partial(_gat_kernel, heads),
        out_type=[
            jax.ShapeDtypeStruct((NP, TW), F32),
            jax.ShapeDtypeStruct((NP, TW), F32),
        ],
        mesh=mesh,
        compiler_params=pltpu.CompilerParams(use_tc_tiling_on_sc=False),
        scratch_types=[
            pltpu.VMEM((CH,), jnp.int32),
            pltpu.VMEM((CH,), jnp.int32),
            pltpu.VMEM((CH,), jnp.int32),
            pltpu.VMEM((CH,), jnp.int32),
            pltpu.VMEM((CH,), jnp.int32),
            pltpu.VMEM((CH,), jnp.int32),
            pltpu.VMEM((CH, TW), F32),
            pltpu.VMEM((CH, TW), F32),
            pltpu.VMEM((CH, 16), F32),
            pltpu.VMEM((CH, 16), F32),
            pltpu.VMEM((CH, TW), F32),
            pltpu.VMEM((CH, TW), F32),
            pltpu.VMEM_SHARED((AR, TW), F32),
            pltpu.SemaphoreType.DMA,
            pltpu.SemaphoreType.DMA,
            pltpu.SemaphoreType.DMA,
            pltpu.SemaphoreType.DMA,
            pltpu.SemaphoreType.DMA,
            pltpu.SemaphoreType.DMA,
        ],
    )
    return kfn(srcp, dstp, tbl, rtbl)


def kernel(x, edge_index, fa_w1, fa_b1, fa_w2, fa_b2, mp_w, mp_b,
           g1_w, g1_al, g1_ar, g1_b, g2_w, g2_al, g2_ar, g2_b,
           agg_w, agg_b, c_w1, c_b1, c_w2, c_b2):
    xp = jnp.pad(x, ((0, NP - N), (0, 0)))
    src = jnp.concatenate([edge_index[0],
                           jnp.zeros((EPA - E,), jnp.int32)])
    dst = jnp.concatenate([edge_index[1],
                           jnp.full((EP - E,), N, jnp.int32),
                           jnp.zeros((EPA - EP,), jnp.int32)])

    eye4 = jnp.eye(4, 16, dtype=F32)
    wl1 = (g1_al[:, :, None] * eye4[:, None, :]).reshape(HID, 16)
    wr1 = (g1_ar[:, :, None] * eye4[:, None, :]).reshape(HID, 16)
    eye1 = jnp.eye(1, 16, dtype=F32)
    wl2 = (g2_al[:, :, None] * eye1[:, None, :]).reshape(HID, 16)
    wr2 = (g2_ar[:, :, None] * eye1[:, None, :]).reshape(HID, 16)

    bh1 = jnp.concatenate(
        [jnp.repeat(jnp.eye(4, dtype=F32), 32, axis=1),
         jnp.zeros((12, HID), F32)], axis=0)
    bh2 = jnp.concatenate(
        [jnp.ones((1, HID), F32), jnp.zeros((15, HID), F32)], axis=0)

    r2 = lambda v: v.reshape(1, -1)

    f, ta, tb, tc, td = _tc1(xp, fa_w1, r2(fa_b1), fa_w2, r2(fa_b2))

    sa, sb, sc, sd = _sc_moments(src, dst, ta, tb, tc, td)

    t1, r1 = _tc2(sa, sb, sc, sd, f, mp_w, r2(mp_b), g1_w, wl1, wr1)

    p0, p1 = _sc_gat(4, src, dst, t1, r1)

    x1, t2, r2t = _tc3(p0, p1, bh1, r2(g1_b), g2_w, wl2, wr2)

    q0, q1 = _sc_gat(1, src, dst, t2, r2t)

    logits = _tc4(q0, q1, x1, bh2, r2(g2_b), agg_w[:HID], agg_w[HID:],
                  r2(agg_b), c_w1, r2(c_b1), c_w2, r2(c_b2))

    return logits[:N]


# speedup vs baseline: 22.8880x; 1.0006x over previous
import functools

import jax
import jax.numpy as jnp
from jax import lax
from jax.experimental import pallas as pl
from jax.experimental.pallas import tpu as pltpu
from jax.experimental.pallas import tpu_sc as plsc

N = 10000
NP = 10240
AR = 10016
E = 320000
EP = 327680
EPA = EP + 256
CHM = 128
CH = 64
D = 128
HID = 128
OUT = 64
MW = 104
TW = 144
NC, NS = 2, 16

_BLK = 1024
_GRID = NP // _BLK

F32 = jnp.float32


def _dot(a, b):
    return jax.lax.dot_general(a, b, (((1,), (0,)), ((), ())),
                               precision=jax.lax.Precision.HIGHEST,
                               preferred_element_type=F32)


def _tc1_body(x_ref, w1_ref, b1_ref, w2_ref, b2_ref,
              f_ref, ta_ref, tb_ref, tc_ref, td_ref):
    x = x_ref[...]
    hgate = jnp.maximum(_dot(x, w1_ref[...]) + b1_ref[...], 0.0)
    gates = jax.nn.sigmoid(_dot(hgate, w2_ref[...]) + b2_ref[...])
    f = x * gates
    f2 = f * f
    f3 = f2 * f
    ones1 = jnp.ones((x.shape[0], 1), F32)
    z31 = jnp.zeros((x.shape[0], 31), F32)
    f_ref[...] = f
    ta_ref[...] = f[:, :104]
    tb_ref[...] = jnp.concatenate([f[:, 104:], f2[:, :80]], axis=1)
    tc_ref[...] = jnp.concatenate([f2[:, 80:], f3[:, :56]], axis=1)
    td_ref[...] = jnp.concatenate([f3[:, 56:], ones1, z31], axis=1)


def _tc1(x, fa_w1, fa_b1, fa_w2, fa_b2):
    row = lambda i: (i, 0)
    full = lambda i: (0, 0)
    mom = pl.BlockSpec((_BLK, MW), row)
    return pl.pallas_call(
        _tc1_body,
        grid=(_GRID,),
        in_specs=[
            pl.BlockSpec((_BLK, D), row),
            pl.BlockSpec((D, 32), full),
            pl.BlockSpec((1, 32), full),
            pl.BlockSpec((32, D), full),
            pl.BlockSpec((1, D), full),
        ],
        out_specs=[pl.BlockSpec((_BLK, D), row), mom, mom, mom, mom],
        out_shape=[
            jax.ShapeDtypeStruct((NP, D), F32),
            jax.ShapeDtypeStruct((NP, MW), F32),
            jax.ShapeDtypeStruct((NP, MW), F32),
            jax.ShapeDtypeStruct((NP, MW), F32),
            jax.ShapeDtypeStruct((NP, MW), F32),
        ],
    )(x, fa_w1, fa_b1, fa_w2, fa_b2)


def _tc2_body(sa_ref, sb_ref, sc_ref, sd_ref, f_ref, mpw_ref, mpb_ref,
              gw_ref, wl_ref, wr_ref, t_ref, r_ref):
    sa = sa_ref[...]
    sb = sb_ref[...]
    sc = sc_ref[...]
    sd = sd_ref[...]
    f = f_ref[...]
    cnt = sd[:, 72:73]
    d = jnp.maximum(cnt, 1.0)
    m1 = jnp.concatenate([sa, sb[:, :24]], axis=1) / d
    m2 = jnp.concatenate([sb[:, 24:], sc[:, :48]], axis=1) / d
    m3 = jnp.concatenate([sc[:, 48:], sd[:, :72]], axis=1) / d
    var = jnp.maximum(m2 - m1 * m1, 0.0)
    t = var + 1e-6
    skew = (m3 - 3.0 * m1 * m2 + 2.0 * m1 * m1 * m1) / (t * jnp.sqrt(t) + 1e-6)
    mixed = jnp.concatenate([f, m1, var, skew], axis=1)
    h = jnp.maximum(_dot(mixed, mpw_ref[...]) + mpb_ref[...], 0.0)
    hp = _dot(h, gw_ref[...])
    el16 = _dot(hp, wl_ref[...])
    er16 = _dot(hp, wr_ref[...])
    t_ref[...] = jnp.concatenate([hp, el16], axis=1)
    r_ref[...] = er16


def _tc2(sa, sb, sc, sd, f, mp_w, mp_b, g1_w, wl1, wr1):
    row = lambda i: (i, 0)
    full = lambda i: (0, 0)
    mom = pl.BlockSpec((_BLK, MW), row)
    return pl.pallas_call(
        _tc2_body,
        grid=(_GRID,),
        in_specs=[
            mom, mom, mom, mom,
            pl.BlockSpec((_BLK, D), row),
            pl.BlockSpec((4 * D, HID), full),
            pl.BlockSpec((1, HID), full),
            pl.BlockSpec((HID, HID), full),
            pl.BlockSpec((HID, 16), full),
            pl.BlockSpec((HID, 16), full),
        ],
        out_specs=[
            pl.BlockSpec((_BLK, TW), row),
            pl.BlockSpec((_BLK, 16), row),
        ],
        out_shape=[
            jax.ShapeDtypeStruct((NP, TW), F32),
            jax.ShapeDtypeStruct((NP, 16), F32),
        ],
    )(sa, sb, sc, sd, f, mp_w, mp_b, g1_w, wl1, wr1)


def _tc3_body(p0_ref, p1_ref, bh_ref, g1b_ref, gw_ref, wl_ref, wr_ref,
              x1_ref, t_ref, r_ref):
    p = p0_ref[...] + p1_ref[...]
    den = _dot(p[:, :16], bh_ref[...])
    x1 = p[:, 16:] / (den + 1e-9) + g1b_ref[...]
    x1 = jnp.where(x1 > 0, x1, jnp.exp(x1) - 1.0)
    hp = _dot(x1, gw_ref[...])
    el16 = _dot(hp, wl_ref[...])
    er16 = _dot(hp, wr_ref[...])
    x1_ref[...] = x1
    t_ref[...] = jnp.concatenate([hp, el16], axis=1)
    r_ref[...] = er16


def _tc3(p0, p1, bh1, g1_b, g2_w, wl2, wr2):
    row = lambda i: (i, 0)
    full = lambda i: (0, 0)
    return pl.pallas_call(
        _tc3_body,
        grid=(_GRID,),
        in_specs=[
            pl.BlockSpec((_BLK, TW), row),
            pl.BlockSpec((_BLK, TW), row),
            pl.BlockSpec((16, HID), full),
            pl.BlockSpec((1, HID), full),
            pl.BlockSpec((HID, HID), full),
            pl.BlockSpec((HID, 16), full),
            pl.BlockSpec((HID, 16), full),
        ],
        out_specs=[
            pl.BlockSpec((_BLK, HID), row),
            pl.BlockSpec((_BLK, TW), row),
            pl.BlockSpec((_BLK, 16), row),
        ],
        out_shape=[
            jax.ShapeDtypeStruct((NP, HID), F32),
            jax.ShapeDtypeStruct((NP, TW), F32),
            jax.ShapeDtypeStruct((NP, 16), F32),
        ],
    )(p0, p1, bh1, g1_b, g2_w, wl2, wr2)


def _tc4_body(q0_ref, q1_ref, x1_ref, bh_ref, g2b_ref, aw1_ref, aw2_ref,
              ab_ref, cw1_ref, cb1_ref, cw2_ref, cb2_ref, out_ref):
    q = q0_ref[...] + q1_ref[...]
    den = _dot(q[:, :16], bh_ref[...])
    x2 = q[:, 16:] / (den + 1e-9) + g2b_ref[...]
    x2 = jnp.where(x2 > 0, x2, jnp.exp(x2) - 1.0)
    x1 = x1_ref[...]
    agg = jnp.maximum(_dot(x1, aw1_ref[...]) + _dot(x2, aw2_ref[...])
                      + ab_ref[...], 0.0)
    hc = jnp.maximum(_dot(agg, cw1_ref[...]) + cb1_ref[...], 0.0)
    out_ref[...] = _dot(hc, cw2_ref[...]) + cb2_ref[...]


def _tc4(q0, q1, x1, bh2, g2_b, aw1, aw2, agg_b, c_w1, c_b1, c_w2, c_b2):
    row = lambda i: (i, 0)
    full = lambda i: (0, 0)
    return pl.pallas_call(
        _tc4_body,
        grid=(_GRID,),
        in_specs=[
            pl.BlockSpec((_BLK, TW), row),
            pl.BlockSpec((_BLK, TW), row),
            pl.BlockSpec((_BLK, HID), row),
            pl.BlockSpec((16, HID), full),
            pl.BlockSpec((1, HID), full),
            pl.BlockSpec((HID, HID), full),
            pl.BlockSpec((HID, HID), full),
            pl.BlockSpec((1, HID), full),
            pl.BlockSpec((HID, HID), full),
            pl.BlockSpec((1, HID), full),
            pl.BlockSpec((HID, OUT), full),
            pl.BlockSpec((1, OUT), full),
        ],
        out_specs=[pl.BlockSpec((_BLK, OUT), row)],
        out_shape=[jax.ShapeDtypeStruct((NP, OUT), F32)],
    )(q0, q1, x1, bh2, g2_b, aw1, aw2, agg_b, c_w1, c_b1, c_w2, c_b2)[0]


def _zero_vmem(buf, rows, cols):
    z = jnp.zeros((16,), F32)
    offs = [j * 16 for j in range(cols // 16)]
    if cols % 16:
        offs.append(cols - 16)

    def row_body(r, _):
        for o in offs:
            buf[r, pl.ds(o, 16)] = z
        return 0

    lax.fori_loop(0, rows, row_body, 0)


def _zero_acc(acc, zbuf, wid, zr):
    nchk = AR // zr
    rem = AR - nchk * zr
    for i in range(-(-nchk // NS)):
        cidx = wid + i * NS

        @pl.when(cidx < nchk)
        def _():
            pltpu.sync_copy(zbuf, acc.at[pl.ds(cidx * zr, zr), :])

    if rem:
        @pl.when(wid == 0)
        def _():
            pltpu.sync_copy(zbuf.at[pl.ds(0, rem), :],
                            acc.at[pl.ds(nchk * zr, rem), :])


def _copy_out(acc, out_ref, wid, zr):
    nchk = AR // zr
    rem = AR - nchk * zr
    for i in range(-(-nchk // NS)):
        cidx = wid + i * NS

        @pl.when(cidx < nchk)
        def _():
            pltpu.sync_copy(acc.at[pl.ds(cidx * zr, zr), :],
                            out_ref.at[pl.ds(cidx * zr, zr), :])

    if rem:
        @pl.when(wid == 0)
        def _():
            pltpu.sync_copy(acc.at[pl.ds(nchk * zr, rem), :],
                            out_ref.at[pl.ds(nchk * zr, rem), :])


def _copy_idx(dst, src):
    for j in range(src.shape[0] // 16):
        dst[pl.ds(j * 16, 16)] = src[pl.ds(j * 16, 16)]


def _moment_kernel(srcp, dstp, ta, tb, tc, td, sa, sb, sc, sd,
                   i_s0, i_s1, i_s2, i_s3, i_d0, i_d1, i_d2, i_d3,
                   sx0, sx1, sx2, sx3, r0, r1, r2_, r3, acc,
                   si0, si1, si2, si3, sg0, sg1, sg2, sg3,
                   ss0, ss1, ss2, ss3):
    idx_s = (i_s0, i_s1, i_s2, i_s3)
    idx_d = (i_d0, i_d1, i_d2, i_d3)
    sidx = (sx0, sx1, sx2, sx3)
    rows = (r0, r1, r2_, r3)
    si = (si0, si1, si2, si3)
    sg = (sg0, sg1, sg2, sg3)
    ss = (ss0, ss1, ss2, ss3)

    cid = lax.axis_index("c")
    wid = lax.axis_index("s")
    per_sub = EP // NS
    nch = per_sub // CHM
    base = wid * per_sub

    def idx_wait(q):
        pltpu.make_async_copy(srcp.at[pl.ds(0, CHM)], idx_s[q], si[q]).wait()
        pltpu.make_async_copy(srcp.at[pl.ds(0, CHM)], idx_d[q], si[q]).wait()

    def idx_load(c, q):
        off = base + c * CHM
        pltpu.async_copy(srcp.at[pl.ds(off, CHM)], idx_s[q], si[q])
        pltpu.async_copy(dstp.at[pl.ds(off, CHM)], idx_d[q], si[q])

    def one_pass(t0, t1, o0, o1):
        _zero_vmem(rows[0], CHM, MW)
        _zero_acc(acc, rows[0], wid, CHM)
        plsc.subcore_barrier()

        def gather(q):
            @pl.when(cid == 0)
            def _():
                pltpu.async_copy(t0.at[idx_s[q]], rows[q], sg[q])

            @pl.when(cid == 1)
            def _():
                pltpu.async_copy(t1.at[idx_s[q]], rows[q], sg[q])

        def scat_wait(q):
            pltpu.make_async_copy(rows[q], acc.at[sidx[q]], ss[q]).wait()

        idx_load(0, 0)
        idx_load(1, 1)
        idx_wait(0)
        gather(0)

        def half(k, j):
            c = 4 * k + j
            qn = (j + 1) % 4
            if j == 3:
                scat_wait(qn)
            else:
                @pl.when(k > 0)
                def _():
                    scat_wait(qn)

            idx_wait(qn)
            gather(qn)

            pltpu.make_async_copy(t0.at[idx_s[j]], rows[j], sg[j]).wait()
            _copy_idx(sidx[j], idx_d[j])
            idx_load(c + 2, (j + 2) % 4)

            pltpu.async_copy(rows[j], acc.at[sidx[j]], ss[j], add=True)

        def body(k, _):
            for j in range(4):
                half(k, j)
            return 0

        lax.fori_loop(0, nch // 4, body, 0)

        scat_wait(1)
        scat_wait(2)
        scat_wait(3)
        pltpu.make_async_copy(t0.at[idx_s[0]], rows[0], sg[0]).wait()
        idx_wait(1)
        plsc.subcore_barrier()

        @pl.when(cid == 0)
        def _():
            _copy_out(acc, o0, wid, CHM)

        @pl.when(cid == 1)
        def _():
            _copy_out(acc, o1, wid, CHM)

        plsc.subcore_barrier()

    one_pass(ta, tc, sa, sc)
    one_pass(tb, td, sb, sd)


def _sc_moments(srcp, dstp, ta, tb, tc, td):
    mesh = plsc.VectorSubcoreMesh(core_axis_name="c", subcore_axis_name="s")
    out = jax.ShapeDtypeStruct((NP, MW), F32)
    kfn = pl.kernel(
        _moment_kernel,
        out_type=[out, out, out, out],
        mesh=mesh,
        compiler_params=pltpu.CompilerParams(use_tc_tiling_on_sc=False),
        scratch_types=(
            [pltpu.VMEM((CHM,), jnp.int32)] * 12
            + [pltpu.VMEM((CHM, MW), F32)] * 4
            + [pltpu.VMEM_SHARED((AR, MW), F32)]
            + [pltpu.SemaphoreType.DMA] * 12
        ),
    )
    return kfn(srcp, dstp, ta, tb, tc, td)


def _gat_kernel(heads, srcp, dstp, tbl, rtbl, p0_out, p1_out,
                idx_s0, idx_s1, idx_d0, idx_d1, sidx0, sidx1,
                rbuf0, rbuf1, ebuf0, ebuf1, obuf0, obuf1, acc,
                si0, si1, sg0, sg1, ss0, ss1):
    cid = lax.axis_index("c")
    wid = lax.axis_index("s")

    _zero_vmem(obuf0, CH, TW)
    _zero_acc(acc, obuf0, wid, CH)
    plsc.subcore_barrier()

    w32 = wid * NC + cid
    per_w = EP // (NC * NS)
    nch = per_w // CH
    base = w32 * per_w
    bph = (128 // heads) // 16

    def idx_wait(sem, dst):
        pltpu.make_async_copy(srcp.at[pl.ds(0, CH)], dst, sem).wait()

    def compute(rbuf, ebuf, obuf):
        def edge(e, _):
            vel = rbuf[e, pl.ds(128, 16)]
            ver = ebuf[e, pl.ds(0, 16)]
            s = vel + ver
            w = jnp.exp(jnp.where(s > 0, s, 0.2 * s))
            obuf[e, pl.ds(0, 16)] = w
            for h in range(heads):
                wh = w[h]
                for b in range(bph):
                    c0 = h * (128 // heads) + b * 16
                    obuf[e, pl.ds(16 + c0, 16)] = rbuf[e, pl.ds(c0, 16)] * wh
            return 0

        lax.fori_loop(0, CH, edge, 0)

    bufs = ((idx_s0, idx_d0, sidx0, rbuf0, ebuf0, obuf0, si0, sg0, ss0),
            (idx_s1, idx_d1, sidx1, rbuf1, ebuf1, obuf1, si1, sg1, ss1))

    def half(k, c, b):
        idx_s, idx_d, sidx, rbuf, ebuf, obuf, si, sg, ss = bufs[b]
        nidx_s, nidx_d, _, nrbuf, nebuf, _, nsi, nsg, _ = bufs[1 - b]
        idx_wait(nsi, nidx_s)
        idx_wait(nsi, nidx_d)
        pltpu.async_copy(tbl.at[nidx_s], nrbuf, nsg)
        pltpu.async_copy(rtbl.at[nidx_d], nebuf, nsg)

        pltpu.make_async_copy(tbl.at[idx_s], rbuf, sg).wait()
        pltpu.make_async_copy(rtbl.at[idx_d], ebuf, sg).wait()

        @pl.when(k > 0)
        def _():
            pltpu.make_async_copy(obuf, acc.at[sidx], ss).wait()

        _copy_idx(sidx, idx_d)
        off2 = base + (c + 2) * CH
        pltpu.async_copy(srcp.at[pl.ds(off2, CH)], idx_s, si)
        pltpu.async_copy(dstp.at[pl.ds(off2, CH)], idx_d, si)

        compute(rbuf, ebuf, obuf)
        pltpu.async_copy(obuf, acc.at[sidx], ss, add=True)

    pltpu.async_copy(srcp.at[pl.ds(base, CH)], idx_s0, si0)
    pltpu.async_copy(dstp.at[pl.ds(base, CH)], idx_d0, si0)
    pltpu.async_copy(srcp.at[pl.ds(base + CH, CH)], idx_s1, si1)
    pltpu.async_copy(dstp.at[pl.ds(base + CH, CH)], idx_d1, si1)
    idx_wait(si0, idx_s0)
    idx_wait(si0, idx_d0)
    pltpu.async_copy(tbl.at[idx_s0], rbuf0, sg0)
    pltpu.async_copy(rtbl.at[idx_d0], ebuf0, sg0)

    def body(k, _):
        half(k, 2 * k, 0)
        half(k, 2 * k + 1, 1)
        return 0

    lax.fori_loop(0, nch // 2, body, 0)

    pltpu.make_async_copy(tbl.at[idx_s0], rbuf0, sg0).wait()
    pltpu.make_async_copy(rtbl.at[idx_d0], ebuf0, sg0).wait()
    pltpu.make_async_copy(obuf0, acc.at[sidx0], ss0).wait()
    pltpu.make_async_copy(obuf1, acc.at[sidx1], ss1).wait()
    idx_wait(si1, idx_s1)
    idx_wait(si1, idx_d1)
    plsc.subcore_barrier()

    @pl.when(cid == 0)
    def _():
        _copy_out(acc, p0_out, wid, CH)

    @pl.when(cid == 1)
    def _():
        _copy_out(acc, p1_out, wid, CH)


def _sc_gat(heads, srcp, dstp, tbl, rtbl):
    mesh = plsc.VectorSubcoreMesh(core_axis_name="c", subcore_axis_name="s")
    kfn = pl.kernel(
        functools.partial(_gat_kernel, heads),
        out_type=[
            jax.ShapeDtypeStruct((NP, TW), F32),
            jax.ShapeDtypeStruct((NP, TW), F32),
        ],
        mesh=mesh,
        compiler_params=pltpu.CompilerParams(use_tc_tiling_on_sc=False),
        scratch_types=[
            pltpu.VMEM((CH,), jnp.int32),
            pltpu.VMEM((CH,), jnp.int32),
            pltpu.VMEM((CH,), jnp.int32),
            pltpu.VMEM((CH,), jnp.int32),
            pltpu.VMEM((CH,), jnp.int32),
            pltpu.VMEM((CH,), jnp.int32),
            pltpu.VMEM((CH, TW), F32),
            pltpu.VMEM((CH, TW), F32),
            pltpu.VMEM((CH, 16), F32),
            pltpu.VMEM((CH, 16), F32),
            pltpu.VMEM((CH, TW), F32),
            pltpu.VMEM((CH, TW), F32),
            pltpu.VMEM_SHARED((AR, TW), F32),
            pltpu.SemaphoreType.DMA,
            pltpu.SemaphoreType.DMA,
            pltpu.SemaphoreType.DMA,
            pltpu.SemaphoreType.DMA,
            pltpu.SemaphoreType.DMA,
            pltpu.SemaphoreType.DMA,
        ],
    )
    return kfn(srcp, dstp, tbl, rtbl)


def kernel(x, edge_index, fa_w1, fa_b1, fa_w2, fa_b2, mp_w, mp_b,
           g1_w, g1_al, g1_ar, g1_b, g2_w, g2_al, g2_ar, g2_b,
           agg_w, agg_b, c_w1, c_b1, c_w2, c_b2):
    xp = jnp.pad(x, ((0, NP - N), (0, 0)))
    src = jnp.concatenate([edge_index[0],
                           jnp.zeros((EPA - E,), jnp.int32)])
    dst = jnp.concatenate([edge_index[1],
                           jnp.full((EP - E,), N, jnp.int32),
                           jnp.zeros((EPA - EP,), jnp.int32)])

    eye4 = jnp.eye(4, 16, dtype=F32)
    wl1 = (g1_al[:, :, None] * eye4[:, None, :]).reshape(HID, 16)
    wr1 = (g1_ar[:, :, None] * eye4[:, None, :]).reshape(HID, 16)
    eye1 = jnp.eye(1, 16, dtype=F32)
    wl2 = (g2_al[:, :, None] * eye1[:, None, :]).reshape(HID, 16)
    wr2 = (g2_ar[:, :, None] * eye1[:, None, :]).reshape(HID, 16)

    bh1 = jnp.concatenate(
        [jnp.repeat(jnp.eye(4, dtype=F32), 32, axis=1),
         jnp.zeros((12, HID), F32)], axis=0)
    bh2 = jnp.concatenate(
        [jnp.ones((1, HID), F32), jnp.zeros((15, HID), F32)], axis=0)

    r2 = lambda v: v.reshape(1, -1)

    f, ta, tb, tc, td = _tc1(xp, fa_w1, r2(fa_b1), fa_w2, r2(fa_b2))

    sa, sb, sc, sd = _sc_moments(src, dst, ta, tb, tc, td)

    t1, r1 = _tc2(sa, sb, sc, sd, f, mp_w, r2(mp_b), g1_w, wl1, wr1)

    p0, p1 = _sc_gat(4, src, dst, t1, r1)

    x1, t2, r2t = _tc3(p0, p1, bh1, r2(g1_b), g2_w, wl2, wr2)

    q0, q1 = _sc_gat(1, src, dst, t2, r2t)

    logits = _tc4(q0, q1, x1, bh2, r2(g2_b), agg_w[:HID], agg_w[HID:],
                  r2(agg_b), c_w1, r2(c_b1), c_w2, r2(c_b2))

    return logits[:N]
